# Initial kernel scaffold; baseline (speedup 1.0000x reference)
import jax
import jax.numpy as jnp
from jax.experimental import pallas as pl


def kernel(pos, h, edge_attr, node_mask, node_time_emb, edge_time_emb, edge_emb_w, edge_emb_b, n2e_w, n2e_b, wq, bq, wk, bk, wv, bv, we, be, wo, bo, ff1_w, ff1_b, ff2_w, ff2_b, ff3_w, ff3_b, ff4_w, ff4_b, ntime_w, ntime_b, etime_w, etime_b, eq_scale, eq_time_w, eq_time_b, eq_in_w, eq_in_b, eq_c1_w, eq_c1_b, eq_c2_w, edge_index):
    raise NotImplementedError("write your pallas kernel here")


# speedup vs baseline: 11.3378x; 11.3378x over previous
import jax
import jax.numpy as jnp
from jax import lax
from jax.experimental import pallas as pl
from jax.experimental.pallas import tpu as pltpu
from jax.experimental.pallas import tpu_sc as plsc

N = 10000
E = 160000
NW = 32
PER_W = E // NW
CK = 120
NFULL = 41
TAIL = PER_W - NFULL * CK
NSUB = 16
RSUB = 624
RTAIL = N - NSUB * RSUB
RN = 1000
RE = 1000

f32 = jnp.float32


def _mesh():
    return plsc.VectorSubcoreMesh(core_axis_name="c", subcore_axis_name="s")


def _make_gather(dims):
    nt = len(dims)
    scratch = []
    for d2 in dims:
        scratch.append(pltpu.VMEM((CK, d2), f32))
        scratch.append(pltpu.VMEM((TAIL, d2), f32))
    scratch += [pltpu.VMEM((CK,), jnp.int32), pltpu.VMEM((TAIL,), jnp.int32),
                pltpu.SemaphoreType.DMA]

    def body(*refs):
        ins = refs[:2 * nt]
        outs = refs[2 * nt:3 * nt]
        bufs = refs[3 * nt:5 * nt]
        idx_v, idxt_v, sem = refs[5 * nt:]
        wid = lax.axis_index("s") * 2 + lax.axis_index("c")
        base = wid * PER_W
        for j in range(nt):
            table, idxa = ins[2 * j], ins[2 * j + 1]
            out = outs[j]
            rows_v, rowst_v = bufs[2 * j], bufs[2 * j + 1]

            def step(i, c, table=table, idxa=idxa, out=out, rows_v=rows_v):
                off = pl.multiple_of(base + i * CK, 8)
                pltpu.sync_copy(idxa.at[pl.ds(off, CK)], idx_v)
                pltpu.async_copy(table.at[idx_v], rows_v, sem).wait()
                pltpu.sync_copy(rows_v, out.at[pl.ds(off, CK)])
                return c

            lax.fori_loop(0, NFULL, step, 0)
            offt = pl.multiple_of(base + NFULL * CK, 8)
            pltpu.sync_copy(idxa.at[pl.ds(offt, TAIL)], idxt_v)
            pltpu.async_copy(table.at[idxt_v], rowst_v, sem).wait()
            pltpu.sync_copy(rowst_v, out.at[pl.ds(offt, TAIL)])

    return body, scratch


def _sc_gather(pairs):
    dims = tuple(int(t.shape[1]) for t, _ in pairs)
    body, scratch = _make_gather(dims)
    outs = tuple(jax.ShapeDtypeStruct((E, d2), f32) for d2 in dims)
    fn = pl.kernel(body, mesh=_mesh(), out_type=outs, scratch_types=scratch)
    flat = []
    for t, ix in pairs:
        flat += [t, ix]
    res = fn(*flat)
    return res if isinstance(res, (tuple, list)) else (res,)


def _sc_scatter_add(vals, idx, zeros):
    w = 128
    scratch = [pltpu.VMEM((CK, w), f32), pltpu.VMEM((TAIL, w), f32),
               pltpu.VMEM((CK,), jnp.int32), pltpu.VMEM((TAIL,), jnp.int32),
               pltpu.VMEM_SHARED((N, w), f32)]

    def body(vals_h, idx_h, zeros_h, out_h, vals_v, valst_v, idx_v, idxt_v, acc):
        cid = lax.axis_index("c")
        sid = lax.axis_index("s")
        wid = sid * 2 + cid
        rs = pl.multiple_of(sid * RSUB, 8)
        pltpu.sync_copy(zeros_h.at[pl.ds(rs, RSUB)], acc.at[pl.ds(rs, RSUB)])

        @pl.when(sid == 0)
        def _():
            pltpu.sync_copy(zeros_h.at[pl.ds(NSUB * RSUB, RTAIL)],
                            acc.at[pl.ds(NSUB * RSUB, RTAIL)])

        plsc.subcore_barrier()
        base = wid * PER_W

        def step(i, c):
            off = pl.multiple_of(base + i * CK, 8)
            pltpu.sync_copy(vals_h.at[pl.ds(off, CK)], vals_v)
            pltpu.sync_copy(idx_h.at[pl.ds(off, CK)], idx_v)
            pltpu.sync_copy(vals_v, acc.at[idx_v], add=True)
            return c

        lax.fori_loop(0, NFULL, step, 0)
        offt = pl.multiple_of(base + NFULL * CK, 8)
        pltpu.sync_copy(vals_h.at[pl.ds(offt, TAIL)], valst_v)
        pltpu.sync_copy(idx_h.at[pl.ds(offt, TAIL)], idxt_v)
        pltpu.sync_copy(valst_v, acc.at[idxt_v], add=True)
        plsc.subcore_barrier()
        off2 = pl.multiple_of(cid * N + sid * RSUB, 8)
        pltpu.sync_copy(acc.at[pl.ds(rs, RSUB)], out_h.at[pl.ds(off2, RSUB)])

        @pl.when(sid == 0)
        def _():
            pltpu.sync_copy(acc.at[pl.ds(NSUB * RSUB, RTAIL)],
                            out_h.at[pl.ds(pl.multiple_of(cid * N + NSUB * RSUB, 8),
                                           RTAIL)])

    fn = pl.kernel(body, mesh=_mesh(),
                   out_type=jax.ShapeDtypeStruct((2 * N, w), f32),
                   scratch_types=scratch)
    return fn(vals, idx, zeros)


def _ln(x):
    mu = jnp.mean(x, axis=-1, keepdims=True)
    var = jnp.mean((x - mu) ** 2, axis=-1, keepdims=True)
    return (x - mu) * lax.rsqrt(var + 1e-6)


def _silu(x):
    return x * (1.0 / (1.0 + jnp.exp(-x)))


def _dm(a, w, b):
    return jnp.dot(a, w[...], preferred_element_type=f32) + b[...]


def _node_pre(h_r, nte_r, w0, b0, w1, b1, w2, b2, w3, b3, w4, b4, w5, b5,
              wq_r, bq_r, wk_r, bk_r, wv_r, bv_r,
              q_o, kv_o, ga_o, shm_o, scm_o, gm_o):
    st = _silu(nte_r[...])
    sh = _dm(st, w0, b0)
    sc = _dm(st, w1, b1)
    hm = _ln(h_r[...]) * (1.0 + sc) + sh
    q_o[...] = _dm(hm, wq_r, bq_r)
    kv_o[:, 0:128] = _dm(hm, wk_r, bk_r)
    kv_o[:, 128:256] = _dm(hm, wv_r, bv_r)
    ga_o[...] = _dm(st, w2, b2)
    shm_o[...] = _dm(st, w3, b3)
    scm_o[...] = _dm(st, w4, b4)
    gm_o[...] = _dm(st, w5, b5)


def _edge_pre(prg, pcg, eat, ete, ewd, ewa, eeb,
              e0, c0, e1, c1, e2, c2, e3, c3, e4, c4, e5, c5,
              we_r, be_r, sfw, sfb, scw, scb, eqs,
              e_o, ega_o, eshm_o, escm_o, egm_o, sft_o, scl_o, geo_o):
    d = prg[...] - pcg[...]
    dist = jnp.sum(d * d, axis=-1, keepdims=True)
    ea = dist * ewd[...] + _dm(eat[...], ewa, eeb)
    st = _silu(ete[...])
    esh = _dm(st, e0, c0)
    esc = _dm(st, e1, c1)
    em = _ln(ea) * (1.0 + esc) + esh
    e_o[...] = _dm(em, we_r, be_r)
    ega_o[...] = _dm(st, e2, c2)
    eshm_o[...] = _dm(st, e3, c3)
    escm_o[...] = _dm(st, e4, c4)
    egm_o[...] = _dm(st, e5, c5)
    sft_o[...] = _dm(st, sfw, sfb)
    scl_o[...] = _dm(st, scw, scb)
    nrm = jnp.sqrt(dist)
    cd = d * (eqs[...] / jnp.maximum(nrm, 1e-8))
    colg = lax.broadcasted_iota(jnp.int32, d.shape, 1)
    geo_o[...] = cd + dist * (colg == 3).astype(f32)


def _attn(qe, kve, ee, ov, ow):
    q = qe[...]
    e = ee[...]
    kj = kve[:, 0:128] + e
    vj = kve[:, 128:256] + e
    s = q * kj
    rM = lax.broadcasted_iota(jnp.int32, (128, 8), 0)
    cM = lax.broadcasted_iota(jnp.int32, (128, 8), 1)
    M = ((rM // 16) == cM).astype(f32)
    w = jnp.exp(jnp.dot(s, M, preferred_element_type=f32) * 0.25)
    rB = lax.broadcasted_iota(jnp.int32, (8, 128), 0)
    cB = lax.broadcasted_iota(jnp.int32, (8, 128), 1)
    B = (rB == (cB // 16)).astype(f32)
    ov[...] = jnp.dot(w, B, preferred_element_type=f32) * vj
    P = (rB == cB).astype(f32)
    ow[...] = jnp.dot(w, P, preferred_element_type=f32)


def _node_post(accv_a, accv_b, accw_a, accw_b, h_r, nmk, ga, shm, scm, gm,
               wo_r, bo_r, f1w, f1b, f2w, f2b, hn_o, ho_o):
    msg = accv_a[...] + accv_b[...]
    aw = accw_a[...] + accw_b[...]
    rD = lax.broadcasted_iota(jnp.int32, (128, 128), 0)
    cD = lax.broadcasted_iota(jnp.int32, (128, 128), 1)
    DP = (rD == (cD // 16)).astype(f32)
    denb = jnp.dot(aw, DP, preferred_element_type=f32)
    hn = _dm(msg / (denb + 1e-16), wo_r, bo_r)
    hn_o[...] = hn
    h1 = h_r[...] + ga[...] * hn
    hmid = (_ln(h1) * (1.0 + scm[...]) + shm[...]) * nmk[...]
    ffn = _dm(_silu(_dm(hmid, f1w, f1b)), f2w, f2b)
    ho_o[...] = (hmid + gm[...] * ffn) * nmk[...]


def _edge_post(hnr, hnc, eat, ega, eshm, escm, egm,
               n2w, n2b, f3w, f3b, f4w, f4b, heo_o):
    he = _dm(hnr[...] + hnc[...], n2w, n2b)
    he1 = eat[...] + ega[...] * he
    he2 = _ln(he1) * (1.0 + escm[...]) + eshm[...]
    ffe = _dm(_silu(_dm(he2, f3w, f3b)), f4w, f4b)
    heo_o[...] = he2 + egm[...] * ffe


def _eqk(hor, hoc, heo, geo, sft, scl,
         wr, wc, we2, wd, eib, c1w, c1b, c2r, o):
    g = geo[...]
    colg = lax.broadcasted_iota(jnp.int32, g.shape, 1)
    dist = jnp.sum(g * (colg == 3).astype(f32), axis=-1, keepdims=True)
    inv = (jnp.dot(hor[...], wr[...], preferred_element_type=f32)
           + jnp.dot(hoc[...], wc[...], preferred_element_type=f32)
           + jnp.dot(heo[...], we2[...], preferred_element_type=f32)
           + dist * wd[...] + eib[...])
    inv = _ln(inv) * (1.0 + scl[...]) + sft[...]
    inv = _dm(_silu(inv), c1w, c1b)
    iv = jnp.tanh(jnp.sum(inv * c2r[...], axis=-1, keepdims=True))
    o[...] = g * (colg < 3).astype(f32) * iv


def _posk(pos_r, acc_a, acc_b, o):
    a2 = acc_a[...] + acc_b[...]
    o[...] = pos_r[...] + a2[:, 0:3]


def _bs(a, r, off=0):
    nd = a.ndim
    return pl.BlockSpec((r,) + a.shape[1:],
                        lambda i, o=off, nd=nd: (i + o,) + (0,) * (nd - 1))


def _bf(a):
    nd = a.ndim
    return pl.BlockSpec(a.shape, lambda i, nd=nd: (0,) * nd)


def _os(w, r):
    return pl.BlockSpec((r, w), lambda i: (i, 0))


def _sd(n, w):
    return jax.ShapeDtypeStruct((n, w), f32)


def kernel(pos, h, edge_attr, node_mask, node_time_emb, edge_time_emb,
           edge_emb_w, edge_emb_b, n2e_w, n2e_b, wq, bq, wk, bk, wv, bv,
           we, be, wo, bo, ff1_w, ff1_b, ff2_w, ff2_b, ff3_w, ff3_b,
           ff4_w, ff4_b, ntime_w, ntime_b, etime_w, etime_b, eq_scale,
           eq_time_w, eq_time_b, eq_in_w, eq_in_b, eq_c1_w, eq_c1_b, eq_c2_w,
           edge_index):
    row = edge_index[0]
    col = edge_index[1]
    pos_pad = jnp.pad(pos, ((0, 0), (0, 125)))

    ntw = [ntime_w[:, i * 128:(i + 1) * 128] for i in range(6)]
    ntb = [ntime_b[i * 128:(i + 1) * 128].reshape(1, -1) for i in range(6)]
    etw = [etime_w[:, i * 16:(i + 1) * 16] for i in range(6)]
    etb = [etime_b[i * 16:(i + 1) * 16].reshape(1, -1) for i in range(6)]
    sfw, scw = eq_time_w[:, 0:128], eq_time_w[:, 128:256]
    sfb = eq_time_b[0:128].reshape(1, -1)
    scb = eq_time_b[128:256].reshape(1, -1)
    ewd = edge_emb_w[0:1, :]
    ewa = edge_emb_w[1:17, :]
    wr_w = eq_in_w[0:128, :]
    wc_w = eq_in_w[128:256, :]
    we2_w = eq_in_w[256:272, :]
    wd_w = eq_in_w[272:273, :]
    c2r = eq_c2_w.reshape(1, 128)
    eqs = eq_scale.reshape(1, 1)
    bqr, bkr, bvr = bq.reshape(1, -1), bk.reshape(1, -1), bv.reshape(1, -1)
    ber, bor = be.reshape(1, -1), bo.reshape(1, -1)
    eebr, n2br = edge_emb_b.reshape(1, -1), n2e_b.reshape(1, -1)
    f1br, f2br = ff1_b.reshape(1, -1), ff2_b.reshape(1, -1)
    f3br, f4br = ff3_b.reshape(1, -1), ff4_b.reshape(1, -1)
    eibr, c1br = eq_in_b.reshape(1, -1), eq_c1_b.reshape(1, -1)

    nbn = N // RN
    nbe = E // RE

    np_full = [ntw[0], ntb[0], ntw[1], ntb[1], ntw[2], ntb[2], ntw[3], ntb[3],
               ntw[4], ntb[4], ntw[5], ntb[5], wq, bqr, wk, bkr, wv, bvr]
    q, kv, n_ga, n_shm, n_scm, n_gm = pl.pallas_call(
        _node_pre, grid=(nbn,),
        in_specs=[_bs(h, RN), _bs(node_time_emb, RN)] + [_bf(x) for x in np_full],
        out_specs=[_os(128, RN), _os(256, RN), _os(128, RN), _os(128, RN),
                   _os(128, RN), _os(128, RN)],
        out_shape=[_sd(N, 128), _sd(N, 256), _sd(N, 128), _sd(N, 128),
                   _sd(N, 128), _sd(N, 128)],
    )(h, node_time_emb, *np_full)

    prg, pcg = _sc_gather([(pos_pad, row), (pos_pad, col)])

    ep_full = [ewd, ewa, eebr, etw[0], etb[0], etw[1], etb[1], etw[2], etb[2],
               etw[3], etb[3], etw[4], etb[4], etw[5], etb[5],
               we, ber, sfw, sfb, scw, scb, eqs]
    e_arr, e_ga, e_shm, e_scm, e_gm, sft, scl, geo = pl.pallas_call(
        _edge_pre, grid=(nbe,),
        in_specs=[_bs(prg, RE), _bs(pcg, RE), _bs(edge_attr, RE),
                  _bs(edge_time_emb, RE)] + [_bf(x) for x in ep_full],
        out_specs=[_os(128, RE), _os(16, RE), _os(16, RE), _os(16, RE),
                   _os(16, RE), _os(128, RE), _os(128, RE), _os(128, RE)],
        out_shape=[_sd(E, 128), _sd(E, 16), _sd(E, 16), _sd(E, 16),
                   _sd(E, 16), _sd(E, 128), _sd(E, 128), _sd(E, 128)],
    )(prg, pcg, edge_attr, edge_time_emb, *ep_full)

    qe, kve = _sc_gather([(q, row), (kv, col)])

    wv_arr, w_arr = pl.pallas_call(
        _attn, grid=(nbe,),
        in_specs=[_bs(qe, RE), _bs(kve, RE), _bs(e_arr, RE)],
        out_specs=[_os(128, RE), _os(128, RE)],
        out_shape=[_sd(E, 128), _sd(E, 128)],
    )(qe, kve, e_arr)

    zeros_n = jnp.zeros((N, 128), f32)
    acc1v = _sc_scatter_add(wv_arr, row, zeros_n)
    acc1w = _sc_scatter_add(w_arr, row, zeros_n)

    npo_full = [wo, bor, ff1_w, f1br, ff2_w, f2br]
    hn, hout = pl.pallas_call(
        _node_post, grid=(nbn,),
        in_specs=[_bs(acc1v, RN), _bs(acc1v, RN, off=nbn),
                  _bs(acc1w, RN), _bs(acc1w, RN, off=nbn), _bs(h, RN),
                  _bs(node_mask, RN), _bs(n_ga, RN), _bs(n_shm, RN),
                  _bs(n_scm, RN), _bs(n_gm, RN)] + [_bf(x) for x in npo_full],
        out_specs=[_os(128, RN), _os(128, RN)],
        out_shape=[_sd(N, 128), _sd(N, 128)],
    )(acc1v, acc1v, acc1w, acc1w, h, node_mask, n_ga, n_shm, n_scm, n_gm,
      *npo_full)

    hnr, hnc = _sc_gather([(hn, row), (hn, col)])

    epo_full = [n2e_w, n2br, ff3_w, f3br, ff4_w, f4br]
    (heo,) = pl.pallas_call(
        _edge_post, grid=(nbe,),
        in_specs=[_bs(hnr, RE), _bs(hnc, RE), _bs(edge_attr, RE),
                  _bs(e_ga, RE), _bs(e_shm, RE), _bs(e_scm, RE),
                  _bs(e_gm, RE)] + [_bf(x) for x in epo_full],
        out_specs=[_os(16, RE)],
        out_shape=[_sd(E, 16)],
    )(hnr, hnc, edge_attr, e_ga, e_shm, e_scm, e_gm, *epo_full)

    hor, hoc = _sc_gather([(hout, row), (hout, col)])

    eq_full = [wr_w, wc_w, we2_w, wd_w, eibr, eq_c1_w, c1br, c2r]
    (scat2,) = pl.pallas_call(
        _eqk, grid=(nbe,),
        in_specs=[_bs(hor, RE), _bs(hoc, RE), _bs(heo, RE), _bs(geo, RE),
                  _bs(sft, RE), _bs(scl, RE)] + [_bf(x) for x in eq_full],
        out_specs=[_os(128, RE)],
        out_shape=[_sd(E, 128)],
    )(hor, hoc, heo, geo, sft, scl, *eq_full)

    acc2 = _sc_scatter_add(scat2, row, zeros_n)

    (pos_out,) = pl.pallas_call(
        _posk, grid=(nbn,),
        in_specs=[_bs(pos, RN), _bs(acc2, RN), _bs(acc2, RN, off=nbn)],
        out_specs=[_os(3, RN)],
        out_shape=[_sd(N, 3)],
    )(pos, acc2, acc2)

    return hout, heo, pos_out


# speedup vs baseline: 12.8205x; 1.1308x over previous
import jax
import jax.numpy as jnp
from jax import lax
from jax.experimental import pallas as pl
from jax.experimental.pallas import tpu as pltpu
from jax.experimental.pallas import tpu_sc as plsc

N = 10000
E = 160000
NW = 32
PER_W = E // NW
CK = 120
NFULL = 41
TAIL = PER_W - NFULL * CK
NSUB = 16
RSUB = 624
RTAIL = N - NSUB * RSUB
RN = 1000
RE = 1000

f32 = jnp.float32


def _mesh():
    return plsc.VectorSubcoreMesh(core_axis_name="c", subcore_axis_name="s")


def _make_gather(dims):
    nt = len(dims)
    scratch = []
    for d2 in dims:
        scratch.append(pltpu.VMEM((CK, d2), f32))
        scratch.append(pltpu.VMEM((CK, d2), f32))
    scratch += [pltpu.VMEM((CK,), jnp.int32), pltpu.VMEM((CK,), jnp.int32),
                pltpu.SemaphoreType.DMA, pltpu.SemaphoreType.DMA,
                pltpu.SemaphoreType.DMA, pltpu.SemaphoreType.DMA,
                pltpu.SemaphoreType.DMA]

    def body(*refs):
        ins = refs[:2 * nt]
        outs = refs[2 * nt:3 * nt]
        bufs = refs[3 * nt:5 * nt]
        idx = refs[5 * nt:5 * nt + 2]
        si = refs[5 * nt + 2:5 * nt + 4]
        ss = refs[5 * nt + 4:5 * nt + 6]
        sg = refs[5 * nt + 6]
        wid = lax.axis_index("s") * 2 + lax.axis_index("c")
        base = wid * PER_W
        for j in range(nt):
            table, idxa = ins[2 * j], ins[2 * j + 1]
            out = outs[j]
            rows = (bufs[2 * j], bufs[2 * j + 1])

            pltpu.async_copy(idxa.at[pl.ds(base, CK)], idx[0], si[0])
            pltpu.async_copy(idxa.at[pl.ds(pl.multiple_of(base + CK, 8), CK)],
                             idx[1], si[1])

            def step(i, carry, table=table, idxa=idxa, out=out, rows=rows):
                for b in (0, 1):
                    c = 2 * i + b
                    off = pl.multiple_of(base + c * CK, 8)

                    @pl.when(c >= 2)
                    def _():
                        pltpu.make_async_copy(
                            rows[b], out.at[pl.ds(off, CK)], ss[b]).wait()

                    pltpu.make_async_copy(
                        idxa.at[pl.ds(off, CK)], idx[b], si[b]).wait()
                    pltpu.async_copy(table.at[idx[b]], rows[b], sg).wait()

                    @pl.when(c + 2 <= NFULL - 1)
                    def _():
                        off2 = pl.multiple_of(base + (c + 2) * CK, 8)
                        pltpu.async_copy(idxa.at[pl.ds(off2, CK)], idx[b], si[b])

                    pltpu.async_copy(rows[b], out.at[pl.ds(off, CK)], ss[b])
                return carry

            lax.fori_loop(0, (NFULL - 1) // 2, step, 0)
            offl = pl.multiple_of(base + (NFULL - 1) * CK, 8)
            pltpu.make_async_copy(rows[0], out.at[pl.ds(offl, CK)], ss[0]).wait()
            pltpu.make_async_copy(idxa.at[pl.ds(offl, CK)], idx[0], si[0]).wait()
            pltpu.async_copy(table.at[idx[0]], rows[0], sg).wait()
            pltpu.async_copy(rows[0], out.at[pl.ds(offl, CK)], ss[0])
            offt = pl.multiple_of(base + NFULL * CK, 8)
            offp = pl.multiple_of(base + (NFULL - 2) * CK, 8)
            pltpu.make_async_copy(rows[1], out.at[pl.ds(offp, CK)], ss[1]).wait()
            pltpu.sync_copy(idxa.at[pl.ds(offt, TAIL)], idx[1].at[pl.ds(0, TAIL)])
            pltpu.async_copy(table.at[idx[1].at[pl.ds(0, TAIL)]],
                             rows[1].at[pl.ds(0, TAIL)], sg).wait()
            pltpu.sync_copy(rows[1].at[pl.ds(0, TAIL)], out.at[pl.ds(offt, TAIL)])
            pltpu.make_async_copy(rows[0], out.at[pl.ds(offl, CK)], ss[0]).wait()

    return body, scratch


def _sc_gather(pairs):
    dims = tuple(int(t.shape[1]) for t, _ in pairs)
    body, scratch = _make_gather(dims)
    outs = tuple(jax.ShapeDtypeStruct((E, d2), f32) for d2 in dims)
    fn = pl.kernel(body, mesh=_mesh(), out_type=outs, scratch_types=scratch)
    flat = []
    for t, ix in pairs:
        flat += [t, ix]
    res = fn(*flat)
    return res if isinstance(res, (tuple, list)) else (res,)


def _sc_scatter_add(vals, idx, zeros):
    w = 128
    scratch = [pltpu.VMEM((CK, w), f32), pltpu.VMEM((CK, w), f32),
               pltpu.VMEM((TAIL, w), f32),
               pltpu.VMEM((CK,), jnp.int32), pltpu.VMEM((CK,), jnp.int32),
               pltpu.VMEM((TAIL,), jnp.int32),
               pltpu.VMEM_SHARED((N, w), f32),
               pltpu.SemaphoreType.DMA, pltpu.SemaphoreType.DMA,
               pltpu.SemaphoreType.DMA, pltpu.SemaphoreType.DMA]

    def body(vals_h, idx_h, zeros_h, out_h, vals0, vals1, valst_v,
             idx0, idx1, idxt_v, acc, sv0, sv1, si0, si1):
        vals = (vals0, vals1)
        idx = (idx0, idx1)
        sv = (sv0, sv1)
        si = (si0, si1)
        cid = lax.axis_index("c")
        sid = lax.axis_index("s")
        wid = sid * 2 + cid
        rs = pl.multiple_of(sid * RSUB, 8)
        pltpu.sync_copy(zeros_h.at[pl.ds(rs, RSUB)], acc.at[pl.ds(rs, RSUB)])

        @pl.when(sid == 0)
        def _():
            pltpu.sync_copy(zeros_h.at[pl.ds(NSUB * RSUB, RTAIL)],
                            acc.at[pl.ds(NSUB * RSUB, RTAIL)])

        plsc.subcore_barrier()
        base = wid * PER_W

        for b in (0, 1):
            offp = pl.multiple_of(base + b * CK, 8)
            pltpu.async_copy(vals_h.at[pl.ds(offp, CK)], vals[b], sv[b])
            pltpu.async_copy(idx_h.at[pl.ds(offp, CK)], idx[b], si[b])

        def step(i, carry):
            for b in (0, 1):
                c = 2 * i + b
                off = pl.multiple_of(base + c * CK, 8)
                pltpu.make_async_copy(
                    vals_h.at[pl.ds(off, CK)], vals[b], sv[b]).wait()
                pltpu.make_async_copy(
                    idx_h.at[pl.ds(off, CK)], idx[b], si[b]).wait()
                pltpu.sync_copy(vals[b], acc.at[idx[b]], add=True)

                @pl.when(c + 2 <= NFULL - 1)
                def _():
                    off2 = pl.multiple_of(base + (c + 2) * CK, 8)
                    pltpu.async_copy(vals_h.at[pl.ds(off2, CK)], vals[b], sv[b])
                    pltpu.async_copy(idx_h.at[pl.ds(off2, CK)], idx[b], si[b])
            return carry

        lax.fori_loop(0, (NFULL - 1) // 2, step, 0)
        offl = pl.multiple_of(base + (NFULL - 1) * CK, 8)
        pltpu.make_async_copy(vals_h.at[pl.ds(offl, CK)], vals[0], sv[0]).wait()
        pltpu.make_async_copy(idx_h.at[pl.ds(offl, CK)], idx[0], si[0]).wait()
        pltpu.sync_copy(vals[0], acc.at[idx[0]], add=True)
        offt = pl.multiple_of(base + NFULL * CK, 8)
        pltpu.sync_copy(vals_h.at[pl.ds(offt, TAIL)], valst_v)
        pltpu.sync_copy(idx_h.at[pl.ds(offt, TAIL)], idxt_v)
        pltpu.sync_copy(valst_v, acc.at[idxt_v], add=True)
        plsc.subcore_barrier()
        off2 = pl.multiple_of(cid * N + sid * RSUB, 8)
        pltpu.sync_copy(acc.at[pl.ds(rs, RSUB)], out_h.at[pl.ds(off2, RSUB)])

        @pl.when(sid == 0)
        def _():
            pltpu.sync_copy(acc.at[pl.ds(NSUB * RSUB, RTAIL)],
                            out_h.at[pl.ds(pl.multiple_of(cid * N + NSUB * RSUB, 8),
                                           RTAIL)])

    fn = pl.kernel(body, mesh=_mesh(),
                   out_type=jax.ShapeDtypeStruct((2 * N, w), f32),
                   scratch_types=scratch)
    return fn(vals, idx, zeros)


def _ln(x):
    mu = jnp.mean(x, axis=-1, keepdims=True)
    var = jnp.mean((x - mu) ** 2, axis=-1, keepdims=True)
    return (x - mu) * lax.rsqrt(var + 1e-6)


def _silu(x):
    return x * (1.0 / (1.0 + jnp.exp(-x)))


def _dm(a, w, b):
    return jnp.dot(a, w[...], preferred_element_type=f32) + b[...]


def _node_pre(h_r, nte_r, w0, b0, w1, b1, w2, b2, w3, b3, w4, b4, w5, b5,
              wq_r, bq_r, wk_r, bk_r, wv_r, bv_r,
              q_o, kv_o, ga_o, shm_o, scm_o, gm_o):
    st = _silu(nte_r[...])
    sh = _dm(st, w0, b0)
    sc = _dm(st, w1, b1)
    hm = _ln(h_r[...]) * (1.0 + sc) + sh
    q_o[...] = _dm(hm, wq_r, bq_r)
    kv_o[:, 0:128] = _dm(hm, wk_r, bk_r)
    kv_o[:, 128:256] = _dm(hm, wv_r, bv_r)
    ga_o[...] = _dm(st, w2, b2)
    shm_o[...] = _dm(st, w3, b3)
    scm_o[...] = _dm(st, w4, b4)
    gm_o[...] = _dm(st, w5, b5)


def _edge_pre(prg, pcg, eat, ete, ewd, ewa, eeb,
              e0, c0, e1, c1, e2, c2, e3, c3, e4, c4, e5, c5,
              we_r, be_r, sfw, sfb, scw, scb, eqs,
              e_o, ega_o, eshm_o, escm_o, egm_o, sft_o, scl_o, geo_o):
    d = prg[...] - pcg[...]
    dist = jnp.sum(d * d, axis=-1, keepdims=True)
    ea = dist * ewd[...] + _dm(eat[...], ewa, eeb)
    st = _silu(ete[...])
    esh = _dm(st, e0, c0)
    esc = _dm(st, e1, c1)
    em = _ln(ea) * (1.0 + esc) + esh
    e_o[...] = _dm(em, we_r, be_r)
    ega_o[...] = _dm(st, e2, c2)
    eshm_o[...] = _dm(st, e3, c3)
    escm_o[...] = _dm(st, e4, c4)
    egm_o[...] = _dm(st, e5, c5)
    sft_o[...] = _dm(st, sfw, sfb)
    scl_o[...] = _dm(st, scw, scb)
    nrm = jnp.sqrt(dist)
    cd = d * (eqs[...] / jnp.maximum(nrm, 1e-8))
    colg = lax.broadcasted_iota(jnp.int32, d.shape, 1)
    geo_o[...] = cd + dist * (colg == 3).astype(f32)


def _attn(qe, kve, ee, ov, ow):
    q = qe[...]
    e = ee[...]
    kj = kve[:, 0:128] + e
    vj = kve[:, 128:256] + e
    s = q * kj
    rM = lax.broadcasted_iota(jnp.int32, (128, 8), 0)
    cM = lax.broadcasted_iota(jnp.int32, (128, 8), 1)
    M = ((rM // 16) == cM).astype(f32)
    w = jnp.exp(jnp.dot(s, M, preferred_element_type=f32) * 0.25)
    rB = lax.broadcasted_iota(jnp.int32, (8, 128), 0)
    cB = lax.broadcasted_iota(jnp.int32, (8, 128), 1)
    B = (rB == (cB // 16)).astype(f32)
    ov[...] = jnp.dot(w, B, preferred_element_type=f32) * vj
    P = (rB == cB).astype(f32)
    ow[...] = jnp.dot(w, P, preferred_element_type=f32)


def _node_post(accv_a, accv_b, accw_a, accw_b, h_r, nmk, ga, shm, scm, gm,
               wo_r, bo_r, f1w, f1b, f2w, f2b, hn_o, ho_o):
    msg = accv_a[...] + accv_b[...]
    aw = accw_a[...] + accw_b[...]
    rD = lax.broadcasted_iota(jnp.int32, (128, 128), 0)
    cD = lax.broadcasted_iota(jnp.int32, (128, 128), 1)
    DP = (rD == (cD // 16)).astype(f32)
    denb = jnp.dot(aw, DP, preferred_element_type=f32)
    hn = _dm(msg / (denb + 1e-16), wo_r, bo_r)
    hn_o[...] = hn
    h1 = h_r[...] + ga[...] * hn
    hmid = (_ln(h1) * (1.0 + scm[...]) + shm[...]) * nmk[...]
    ffn = _dm(_silu(_dm(hmid, f1w, f1b)), f2w, f2b)
    ho_o[...] = (hmid + gm[...] * ffn) * nmk[...]


def _edge_post(hnr, hnc, eat, ega, eshm, escm, egm,
               n2w, n2b, f3w, f3b, f4w, f4b, heo_o):
    he = _dm(hnr[...] + hnc[...], n2w, n2b)
    he1 = eat[...] + ega[...] * he
    he2 = _ln(he1) * (1.0 + escm[...]) + eshm[...]
    ffe = _dm(_silu(_dm(he2, f3w, f3b)), f4w, f4b)
    heo_o[...] = he2 + egm[...] * ffe


def _eqk(hor, hoc, heo, geo, sft, scl,
         wr, wc, we2, wd, eib, c1w, c1b, c2r, o):
    g = geo[...]
    colg = lax.broadcasted_iota(jnp.int32, g.shape, 1)
    dist = jnp.sum(g * (colg == 3).astype(f32), axis=-1, keepdims=True)
    inv = (jnp.dot(hor[...], wr[...], preferred_element_type=f32)
           + jnp.dot(hoc[...], wc[...], preferred_element_type=f32)
           + jnp.dot(heo[...], we2[...], preferred_element_type=f32)
           + dist * wd[...] + eib[...])
    inv = _ln(inv) * (1.0 + scl[...]) + sft[...]
    inv = _dm(_silu(inv), c1w, c1b)
    iv = jnp.tanh(jnp.sum(inv * c2r[...], axis=-1, keepdims=True))
    o[...] = g * (colg < 3).astype(f32) * iv


def _posk(pos_r, acc_a, acc_b, o):
    a2 = acc_a[...] + acc_b[...]
    o[...] = pos_r[...] + a2[:, 0:3]


def _bs(a, r, off=0):
    nd = a.ndim
    return pl.BlockSpec((r,) + a.shape[1:],
                        lambda i, o=off, nd=nd: (i + o,) + (0,) * (nd - 1))


def _bf(a):
    nd = a.ndim
    return pl.BlockSpec(a.shape, lambda i, nd=nd: (0,) * nd)


def _os(w, r):
    return pl.BlockSpec((r, w), lambda i: (i, 0))


def _sd(n, w):
    return jax.ShapeDtypeStruct((n, w), f32)


def kernel(pos, h, edge_attr, node_mask, node_time_emb, edge_time_emb,
           edge_emb_w, edge_emb_b, n2e_w, n2e_b, wq, bq, wk, bk, wv, bv,
           we, be, wo, bo, ff1_w, ff1_b, ff2_w, ff2_b, ff3_w, ff3_b,
           ff4_w, ff4_b, ntime_w, ntime_b, etime_w, etime_b, eq_scale,
           eq_time_w, eq_time_b, eq_in_w, eq_in_b, eq_c1_w, eq_c1_b, eq_c2_w,
           edge_index):
    row = edge_index[0]
    col = edge_index[1]
    pos_pad = jnp.pad(pos, ((0, 0), (0, 125)))

    ntw = [ntime_w[:, i * 128:(i + 1) * 128] for i in range(6)]
    ntb = [ntime_b[i * 128:(i + 1) * 128].reshape(1, -1) for i in range(6)]
    etw = [etime_w[:, i * 16:(i + 1) * 16] for i in range(6)]
    etb = [etime_b[i * 16:(i + 1) * 16].reshape(1, -1) for i in range(6)]
    sfw, scw = eq_time_w[:, 0:128], eq_time_w[:, 128:256]
    sfb = eq_time_b[0:128].reshape(1, -1)
    scb = eq_time_b[128:256].reshape(1, -1)
    ewd = edge_emb_w[0:1, :]
    ewa = edge_emb_w[1:17, :]
    wr_w = eq_in_w[0:128, :]
    wc_w = eq_in_w[128:256, :]
    we2_w = eq_in_w[256:272, :]
    wd_w = eq_in_w[272:273, :]
    c2r = eq_c2_w.reshape(1, 128)
    eqs = eq_scale.reshape(1, 1)
    bqr, bkr, bvr = bq.reshape(1, -1), bk.reshape(1, -1), bv.reshape(1, -1)
    ber, bor = be.reshape(1, -1), bo.reshape(1, -1)
    eebr, n2br = edge_emb_b.reshape(1, -1), n2e_b.reshape(1, -1)
    f1br, f2br = ff1_b.reshape(1, -1), ff2_b.reshape(1, -1)
    f3br, f4br = ff3_b.reshape(1, -1), ff4_b.reshape(1, -1)
    eibr, c1br = eq_in_b.reshape(1, -1), eq_c1_b.reshape(1, -1)

    nbn = N // RN
    nbe = E // RE

    np_full = [ntw[0], ntb[0], ntw[1], ntb[1], ntw[2], ntb[2], ntw[3], ntb[3],
               ntw[4], ntb[4], ntw[5], ntb[5], wq, bqr, wk, bkr, wv, bvr]
    q, kv, n_ga, n_shm, n_scm, n_gm = pl.pallas_call(
        _node_pre, grid=(nbn,),
        in_specs=[_bs(h, RN), _bs(node_time_emb, RN)] + [_bf(x) for x in np_full],
        out_specs=[_os(128, RN), _os(256, RN), _os(128, RN), _os(128, RN),
                   _os(128, RN), _os(128, RN)],
        out_shape=[_sd(N, 128), _sd(N, 256), _sd(N, 128), _sd(N, 128),
                   _sd(N, 128), _sd(N, 128)],
    )(h, node_time_emb, *np_full)

    prg, pcg = _sc_gather([(pos_pad, row), (pos_pad, col)])

    ep_full = [ewd, ewa, eebr, etw[0], etb[0], etw[1], etb[1], etw[2], etb[2],
               etw[3], etb[3], etw[4], etb[4], etw[5], etb[5],
               we, ber, sfw, sfb, scw, scb, eqs]
    e_arr, e_ga, e_shm, e_scm, e_gm, sft, scl, geo = pl.pallas_call(
        _edge_pre, grid=(nbe,),
        in_specs=[_bs(prg, RE), _bs(pcg, RE), _bs(edge_attr, RE),
                  _bs(edge_time_emb, RE)] + [_bf(x) for x in ep_full],
        out_specs=[_os(128, RE), _os(16, RE), _os(16, RE), _os(16, RE),
                   _os(16, RE), _os(128, RE), _os(128, RE), _os(128, RE)],
        out_shape=[_sd(E, 128), _sd(E, 16), _sd(E, 16), _sd(E, 16),
                   _sd(E, 16), _sd(E, 128), _sd(E, 128), _sd(E, 128)],
    )(prg, pcg, edge_attr, edge_time_emb, *ep_full)

    qe, kve = _sc_gather([(q, row), (kv, col)])

    wv_arr, w_arr = pl.pallas_call(
        _attn, grid=(nbe,),
        in_specs=[_bs(qe, RE), _bs(kve, RE), _bs(e_arr, RE)],
        out_specs=[_os(128, RE), _os(128, RE)],
        out_shape=[_sd(E, 128), _sd(E, 128)],
    )(qe, kve, e_arr)

    zeros_n = jnp.zeros((N, 128), f32)
    acc1v = _sc_scatter_add(wv_arr, row, zeros_n)
    acc1w = _sc_scatter_add(w_arr, row, zeros_n)

    npo_full = [wo, bor, ff1_w, f1br, ff2_w, f2br]
    hn, hout = pl.pallas_call(
        _node_post, grid=(nbn,),
        in_specs=[_bs(acc1v, RN), _bs(acc1v, RN, off=nbn),
                  _bs(acc1w, RN), _bs(acc1w, RN, off=nbn), _bs(h, RN),
                  _bs(node_mask, RN), _bs(n_ga, RN), _bs(n_shm, RN),
                  _bs(n_scm, RN), _bs(n_gm, RN)] + [_bf(x) for x in npo_full],
        out_specs=[_os(128, RN), _os(128, RN)],
        out_shape=[_sd(N, 128), _sd(N, 128)],
    )(acc1v, acc1v, acc1w, acc1w, h, node_mask, n_ga, n_shm, n_scm, n_gm,
      *npo_full)

    hnr, hnc = _sc_gather([(hn, row), (hn, col)])

    epo_full = [n2e_w, n2br, ff3_w, f3br, ff4_w, f4br]
    (heo,) = pl.pallas_call(
        _edge_post, grid=(nbe,),
        in_specs=[_bs(hnr, RE), _bs(hnc, RE), _bs(edge_attr, RE),
                  _bs(e_ga, RE), _bs(e_shm, RE), _bs(e_scm, RE),
                  _bs(e_gm, RE)] + [_bf(x) for x in epo_full],
        out_specs=[_os(16, RE)],
        out_shape=[_sd(E, 16)],
    )(hnr, hnc, edge_attr, e_ga, e_shm, e_scm, e_gm, *epo_full)

    hor, hoc = _sc_gather([(hout, row), (hout, col)])

    eq_full = [wr_w, wc_w, we2_w, wd_w, eibr, eq_c1_w, c1br, c2r]
    (scat2,) = pl.pallas_call(
        _eqk, grid=(nbe,),
        in_specs=[_bs(hor, RE), _bs(hoc, RE), _bs(heo, RE), _bs(geo, RE),
                  _bs(sft, RE), _bs(scl, RE)] + [_bf(x) for x in eq_full],
        out_specs=[_os(128, RE)],
        out_shape=[_sd(E, 128)],
    )(hor, hoc, heo, geo, sft, scl, *eq_full)

    acc2 = _sc_scatter_add(scat2, row, zeros_n)

    (pos_out,) = pl.pallas_call(
        _posk, grid=(nbn,),
        in_specs=[_bs(pos, RN), _bs(acc2, RN), _bs(acc2, RN, off=nbn)],
        out_specs=[_os(3, RN)],
        out_shape=[_sd(N, 3)],
    )(pos, acc2, acc2)

    return hout, heo, pos_out


# speedup vs baseline: 13.0128x; 1.0150x over previous
import jax
import jax.numpy as jnp
from jax import lax
from jax.experimental import pallas as pl
from jax.experimental.pallas import tpu as pltpu
from jax.experimental.pallas import tpu_sc as plsc

N = 10000
E = 160000
NW = 32
PER_W = E // NW
CK = 120
NFULL = 41
TAIL = PER_W - NFULL * CK
NSUB = 16
RSUB = 624
RTAIL = N - NSUB * RSUB
RN = 1000
RE = 1000

f32 = jnp.float32


def _mesh():
    return plsc.VectorSubcoreMesh(core_axis_name="c", subcore_axis_name="s")


def _make_gather(dims):
    nt = len(dims)
    scratch = []
    for d2 in dims:
        scratch.append(pltpu.VMEM((CK, d2), f32))
        scratch.append(pltpu.VMEM((CK, d2), f32))
    scratch += [pltpu.VMEM((CK,), jnp.int32), pltpu.VMEM((CK,), jnp.int32),
                pltpu.SemaphoreType.DMA, pltpu.SemaphoreType.DMA,
                pltpu.SemaphoreType.DMA, pltpu.SemaphoreType.DMA,
                pltpu.SemaphoreType.DMA, pltpu.SemaphoreType.DMA]

    def body(*refs):
        ins = refs[:2 * nt]
        outs = refs[2 * nt:3 * nt]
        bufs = refs[3 * nt:5 * nt]
        idx = refs[5 * nt:5 * nt + 2]
        si = refs[5 * nt + 2:5 * nt + 4]
        ss = refs[5 * nt + 4:5 * nt + 6]
        sgg = refs[5 * nt + 6:5 * nt + 8]
        wid = lax.axis_index("s") * 2 + lax.axis_index("c")
        base = wid * PER_W
        for j in range(nt):
            table, idxa = ins[2 * j], ins[2 * j + 1]
            out = outs[j]
            rows = (bufs[2 * j], bufs[2 * j + 1])

            pltpu.async_copy(idxa.at[pl.ds(base, CK)], idx[0], si[0])
            pltpu.async_copy(idxa.at[pl.ds(pl.multiple_of(base + CK, 8), CK)],
                             idx[1], si[1])

            def step(i, carry, table=table, idxa=idxa, out=out, rows=rows):
                for b in (0, 1):
                    c = 2 * i + b
                    off = pl.multiple_of(base + c * CK, 8)

                    @pl.when(c >= 2)
                    def _():
                        pltpu.make_async_copy(
                            rows[b], out.at[pl.ds(off, CK)], ss[b]).wait()

                    pltpu.make_async_copy(
                        idxa.at[pl.ds(off, CK)], idx[b], si[b]).wait()
                    pltpu.async_copy(table.at[idx[b]], rows[b], sgg[b])

                    @pl.when(c >= 1)
                    def _():
                        offp = pl.multiple_of(base + (c - 1) * CK, 8)
                        pltpu.make_async_copy(
                            table.at[idx[1 - b]], rows[1 - b], sgg[1 - b]).wait()
                        pltpu.async_copy(rows[1 - b], out.at[pl.ds(offp, CK)],
                                         ss[1 - b])
                        offn = pl.multiple_of(base + (c + 1) * CK, 8)
                        pltpu.async_copy(idxa.at[pl.ds(offn, CK)],
                                         idx[1 - b], si[1 - b])
                return carry

            lax.fori_loop(0, (NFULL - 1) // 2, step, 0)
            offl = pl.multiple_of(base + (NFULL - 1) * CK, 8)
            pltpu.make_async_copy(rows[0], out.at[pl.ds(offl, CK)], ss[0]).wait()
            pltpu.make_async_copy(idxa.at[pl.ds(offl, CK)], idx[0], si[0]).wait()
            pltpu.async_copy(table.at[idx[0]], rows[0], sgg[0])
            offp2 = pl.multiple_of(base + (NFULL - 2) * CK, 8)
            pltpu.make_async_copy(table.at[idx[1]], rows[1], sgg[1]).wait()
            pltpu.async_copy(rows[1], out.at[pl.ds(offp2, CK)], ss[1])
            pltpu.make_async_copy(table.at[idx[0]], rows[0], sgg[0]).wait()
            pltpu.async_copy(rows[0], out.at[pl.ds(offl, CK)], ss[0])
            offt = pl.multiple_of(base + NFULL * CK, 8)
            pltpu.make_async_copy(rows[1], out.at[pl.ds(offp2, CK)], ss[1]).wait()
            pltpu.sync_copy(idxa.at[pl.ds(offt, TAIL)], idx[1].at[pl.ds(0, TAIL)])
            pltpu.async_copy(table.at[idx[1].at[pl.ds(0, TAIL)]],
                             rows[1].at[pl.ds(0, TAIL)], sgg[1]).wait()
            pltpu.sync_copy(rows[1].at[pl.ds(0, TAIL)], out.at[pl.ds(offt, TAIL)])
            pltpu.make_async_copy(rows[0], out.at[pl.ds(offl, CK)], ss[0]).wait()

    return body, scratch


def _sc_gather(pairs):
    dims = tuple(int(t.shape[1]) for t, _ in pairs)
    body, scratch = _make_gather(dims)
    outs = tuple(jax.ShapeDtypeStruct((E, d2), f32) for d2 in dims)
    fn = pl.kernel(body, mesh=_mesh(), out_type=outs, scratch_types=scratch)
    flat = []
    for t, ix in pairs:
        flat += [t, ix]
    res = fn(*flat)
    return res if isinstance(res, (tuple, list)) else (res,)


def _sc_scatter_add(vals, idx, zeros):
    w = 128
    scratch = [pltpu.VMEM((CK, w), f32), pltpu.VMEM((CK, w), f32),
               pltpu.VMEM((TAIL, w), f32),
               pltpu.VMEM((CK,), jnp.int32), pltpu.VMEM((CK,), jnp.int32),
               pltpu.VMEM((TAIL,), jnp.int32),
               pltpu.VMEM_SHARED((N, w), f32),
               pltpu.SemaphoreType.DMA, pltpu.SemaphoreType.DMA,
               pltpu.SemaphoreType.DMA, pltpu.SemaphoreType.DMA]

    def body(vals_h, idx_h, zeros_h, out_h, vals0, vals1, valst_v,
             idx0, idx1, idxt_v, acc, sv0, sv1, si0, si1):
        vals = (vals0, vals1)
        idx = (idx0, idx1)
        sv = (sv0, sv1)
        si = (si0, si1)
        cid = lax.axis_index("c")
        sid = lax.axis_index("s")
        wid = sid * 2 + cid
        rs = pl.multiple_of(sid * RSUB, 8)
        pltpu.sync_copy(zeros_h.at[pl.ds(rs, RSUB)], acc.at[pl.ds(rs, RSUB)])

        @pl.when(sid == 0)
        def _():
            pltpu.sync_copy(zeros_h.at[pl.ds(NSUB * RSUB, RTAIL)],
                            acc.at[pl.ds(NSUB * RSUB, RTAIL)])

        plsc.subcore_barrier()
        base = wid * PER_W

        for b in (0, 1):
            offp = pl.multiple_of(base + b * CK, 8)
            pltpu.async_copy(vals_h.at[pl.ds(offp, CK)], vals[b], sv[b])
            pltpu.async_copy(idx_h.at[pl.ds(offp, CK)], idx[b], si[b])

        def step(i, carry):
            for b in (0, 1):
                c = 2 * i + b
                off = pl.multiple_of(base + c * CK, 8)
                pltpu.make_async_copy(
                    vals_h.at[pl.ds(off, CK)], vals[b], sv[b]).wait()
                pltpu.make_async_copy(
                    idx_h.at[pl.ds(off, CK)], idx[b], si[b]).wait()
                pltpu.sync_copy(vals[b], acc.at[idx[b]], add=True)

                @pl.when(c + 2 <= NFULL - 1)
                def _():
                    off2 = pl.multiple_of(base + (c + 2) * CK, 8)
                    pltpu.async_copy(vals_h.at[pl.ds(off2, CK)], vals[b], sv[b])
                    pltpu.async_copy(idx_h.at[pl.ds(off2, CK)], idx[b], si[b])
            return carry

        lax.fori_loop(0, (NFULL - 1) // 2, step, 0)
        offl = pl.multiple_of(base + (NFULL - 1) * CK, 8)
        pltpu.make_async_copy(vals_h.at[pl.ds(offl, CK)], vals[0], sv[0]).wait()
        pltpu.make_async_copy(idx_h.at[pl.ds(offl, CK)], idx[0], si[0]).wait()
        pltpu.sync_copy(vals[0], acc.at[idx[0]], add=True)
        offt = pl.multiple_of(base + NFULL * CK, 8)
        pltpu.sync_copy(vals_h.at[pl.ds(offt, TAIL)], valst_v)
        pltpu.sync_copy(idx_h.at[pl.ds(offt, TAIL)], idxt_v)
        pltpu.sync_copy(valst_v, acc.at[idxt_v], add=True)
        plsc.subcore_barrier()
        off2 = pl.multiple_of(cid * N + sid * RSUB, 8)
        pltpu.sync_copy(acc.at[pl.ds(rs, RSUB)], out_h.at[pl.ds(off2, RSUB)])

        @pl.when(sid == 0)
        def _():
            pltpu.sync_copy(acc.at[pl.ds(NSUB * RSUB, RTAIL)],
                            out_h.at[pl.ds(pl.multiple_of(cid * N + NSUB * RSUB, 8),
                                           RTAIL)])

    fn = pl.kernel(body, mesh=_mesh(),
                   out_type=jax.ShapeDtypeStruct((2 * N, w), f32),
                   scratch_types=scratch)
    return fn(vals, idx, zeros)


def _ln(x):
    mu = jnp.mean(x, axis=-1, keepdims=True)
    var = jnp.mean((x - mu) ** 2, axis=-1, keepdims=True)
    return (x - mu) * lax.rsqrt(var + 1e-6)


def _silu(x):
    return x * (1.0 / (1.0 + jnp.exp(-x)))


def _dm(a, w, b):
    return jnp.dot(a, w[...], preferred_element_type=f32) + b[...]


def _node_pre(h_r, nte_r, w0, b0, w1, b1, w2, b2, w3, b3, w4, b4, w5, b5,
              wq_r, bq_r, wk_r, bk_r, wv_r, bv_r,
              q_o, kv_o, ga_o, shm_o, scm_o, gm_o):
    st = _silu(nte_r[...])
    sh = _dm(st, w0, b0)
    sc = _dm(st, w1, b1)
    hm = _ln(h_r[...]) * (1.0 + sc) + sh
    q_o[...] = _dm(hm, wq_r, bq_r)
    kv_o[:, 0:128] = _dm(hm, wk_r, bk_r)
    kv_o[:, 128:256] = _dm(hm, wv_r, bv_r)
    ga_o[...] = _dm(st, w2, b2)
    shm_o[...] = _dm(st, w3, b3)
    scm_o[...] = _dm(st, w4, b4)
    gm_o[...] = _dm(st, w5, b5)


def _edge_pre(prg, pcg, eat, ete, ewd, ewa, eeb,
              e0, c0, e1, c1, e2, c2, e3, c3, e4, c4, e5, c5,
              we_r, be_r, sfw, sfb, scw, scb, eqs,
              e_o, ega_o, eshm_o, escm_o, egm_o, sft_o, scl_o, geo_o):
    d = prg[...] - pcg[...]
    dist = jnp.sum(d * d, axis=-1, keepdims=True)
    ea = dist * ewd[...] + _dm(eat[...], ewa, eeb)
    st = _silu(ete[...])
    esh = _dm(st, e0, c0)
    esc = _dm(st, e1, c1)
    em = _ln(ea) * (1.0 + esc) + esh
    e_o[...] = _dm(em, we_r, be_r)
    ega_o[...] = _dm(st, e2, c2)
    eshm_o[...] = _dm(st, e3, c3)
    escm_o[...] = _dm(st, e4, c4)
    egm_o[...] = _dm(st, e5, c5)
    sft_o[...] = _dm(st, sfw, sfb)
    scl_o[...] = _dm(st, scw, scb)
    nrm = jnp.sqrt(dist)
    cd = d * (eqs[...] / jnp.maximum(nrm, 1e-8))
    colg = lax.broadcasted_iota(jnp.int32, d.shape, 1)
    geo_o[...] = cd + dist * (colg == 3).astype(f32)


def _attn(qe, kve, ee, ov, ow):
    q = qe[...]
    e = ee[...]
    kj = kve[:, 0:128] + e
    vj = kve[:, 128:256] + e
    s = q * kj
    rM = lax.broadcasted_iota(jnp.int32, (128, 8), 0)
    cM = lax.broadcasted_iota(jnp.int32, (128, 8), 1)
    M = ((rM // 16) == cM).astype(f32)
    w = jnp.exp(jnp.dot(s, M, preferred_element_type=f32) * 0.25)
    rB = lax.broadcasted_iota(jnp.int32, (8, 128), 0)
    cB = lax.broadcasted_iota(jnp.int32, (8, 128), 1)
    B = (rB == (cB // 16)).astype(f32)
    ov[...] = jnp.dot(w, B, preferred_element_type=f32) * vj
    P = (rB == cB).astype(f32)
    ow[...] = jnp.dot(w, P, preferred_element_type=f32)


def _node_post(accv_a, accv_b, accw_a, accw_b, h_r, nmk, ga, shm, scm, gm,
               wo_r, bo_r, f1w, f1b, f2w, f2b, hn_o, ho_o):
    msg = accv_a[...] + accv_b[...]
    aw = accw_a[...] + accw_b[...]
    rD = lax.broadcasted_iota(jnp.int32, (128, 128), 0)
    cD = lax.broadcasted_iota(jnp.int32, (128, 128), 1)
    DP = (rD == (cD // 16)).astype(f32)
    denb = jnp.dot(aw, DP, preferred_element_type=f32)
    hn = _dm(msg / (denb + 1e-16), wo_r, bo_r)
    hn_o[...] = hn
    h1 = h_r[...] + ga[...] * hn
    hmid = (_ln(h1) * (1.0 + scm[...]) + shm[...]) * nmk[...]
    ffn = _dm(_silu(_dm(hmid, f1w, f1b)), f2w, f2b)
    ho_o[...] = (hmid + gm[...] * ffn) * nmk[...]


def _edge_post(hnr, hnc, eat, ega, eshm, escm, egm,
               n2w, n2b, f3w, f3b, f4w, f4b, heo_o):
    he = _dm(hnr[...] + hnc[...], n2w, n2b)
    he1 = eat[...] + ega[...] * he
    he2 = _ln(he1) * (1.0 + escm[...]) + eshm[...]
    ffe = _dm(_silu(_dm(he2, f3w, f3b)), f4w, f4b)
    heo_o[...] = he2 + egm[...] * ffe


def _eqk(hor, hoc, heo, geo, sft, scl,
         wr, wc, we2, wd, eib, c1w, c1b, c2r, o):
    g = geo[...]
    colg = lax.broadcasted_iota(jnp.int32, g.shape, 1)
    dist = jnp.sum(g * (colg == 3).astype(f32), axis=-1, keepdims=True)
    inv = (jnp.dot(hor[...], wr[...], preferred_element_type=f32)
           + jnp.dot(hoc[...], wc[...], preferred_element_type=f32)
           + jnp.dot(heo[...], we2[...], preferred_element_type=f32)
           + dist * wd[...] + eib[...])
    inv = _ln(inv) * (1.0 + scl[...]) + sft[...]
    inv = _dm(_silu(inv), c1w, c1b)
    iv = jnp.tanh(jnp.sum(inv * c2r[...], axis=-1, keepdims=True))
    o[...] = g * (colg < 3).astype(f32) * iv


def _posk(pos_r, acc_a, acc_b, o):
    a2 = acc_a[...] + acc_b[...]
    o[...] = pos_r[...] + a2[:, 0:3]


def _bs(a, r, off=0):
    nd = a.ndim
    return pl.BlockSpec((r,) + a.shape[1:],
                        lambda i, o=off, nd=nd: (i + o,) + (0,) * (nd - 1))


def _bf(a):
    nd = a.ndim
    return pl.BlockSpec(a.shape, lambda i, nd=nd: (0,) * nd)


def _os(w, r):
    return pl.BlockSpec((r, w), lambda i: (i, 0))


def _sd(n, w):
    return jax.ShapeDtypeStruct((n, w), f32)


def kernel(pos, h, edge_attr, node_mask, node_time_emb, edge_time_emb,
           edge_emb_w, edge_emb_b, n2e_w, n2e_b, wq, bq, wk, bk, wv, bv,
           we, be, wo, bo, ff1_w, ff1_b, ff2_w, ff2_b, ff3_w, ff3_b,
           ff4_w, ff4_b, ntime_w, ntime_b, etime_w, etime_b, eq_scale,
           eq_time_w, eq_time_b, eq_in_w, eq_in_b, eq_c1_w, eq_c1_b, eq_c2_w,
           edge_index):
    row = edge_index[0]
    col = edge_index[1]
    pos_pad = jnp.pad(pos, ((0, 0), (0, 125)))

    ntw = [ntime_w[:, i * 128:(i + 1) * 128] for i in range(6)]
    ntb = [ntime_b[i * 128:(i + 1) * 128].reshape(1, -1) for i in range(6)]
    etw = [etime_w[:, i * 16:(i + 1) * 16] for i in range(6)]
    etb = [etime_b[i * 16:(i + 1) * 16].reshape(1, -1) for i in range(6)]
    sfw, scw = eq_time_w[:, 0:128], eq_time_w[:, 128:256]
    sfb = eq_time_b[0:128].reshape(1, -1)
    scb = eq_time_b[128:256].reshape(1, -1)
    ewd = edge_emb_w[0:1, :]
    ewa = edge_emb_w[1:17, :]
    wr_w = eq_in_w[0:128, :]
    wc_w = eq_in_w[128:256, :]
    we2_w = eq_in_w[256:272, :]
    wd_w = eq_in_w[272:273, :]
    c2r = eq_c2_w.reshape(1, 128)
    eqs = eq_scale.reshape(1, 1)
    bqr, bkr, bvr = bq.reshape(1, -1), bk.reshape(1, -1), bv.reshape(1, -1)
    ber, bor = be.reshape(1, -1), bo.reshape(1, -1)
    eebr, n2br = edge_emb_b.reshape(1, -1), n2e_b.reshape(1, -1)
    f1br, f2br = ff1_b.reshape(1, -1), ff2_b.reshape(1, -1)
    f3br, f4br = ff3_b.reshape(1, -1), ff4_b.reshape(1, -1)
    eibr, c1br = eq_in_b.reshape(1, -1), eq_c1_b.reshape(1, -1)

    nbn = N // RN
    nbe = E // RE

    np_full = [ntw[0], ntb[0], ntw[1], ntb[1], ntw[2], ntb[2], ntw[3], ntb[3],
               ntw[4], ntb[4], ntw[5], ntb[5], wq, bqr, wk, bkr, wv, bvr]
    q, kv, n_ga, n_shm, n_scm, n_gm = pl.pallas_call(
        _node_pre, grid=(nbn,),
        in_specs=[_bs(h, RN), _bs(node_time_emb, RN)] + [_bf(x) for x in np_full],
        out_specs=[_os(128, RN), _os(256, RN), _os(128, RN), _os(128, RN),
                   _os(128, RN), _os(128, RN)],
        out_shape=[_sd(N, 128), _sd(N, 256), _sd(N, 128), _sd(N, 128),
                   _sd(N, 128), _sd(N, 128)],
    )(h, node_time_emb, *np_full)

    prg, pcg = _sc_gather([(pos_pad, row), (pos_pad, col)])

    ep_full = [ewd, ewa, eebr, etw[0], etb[0], etw[1], etb[1], etw[2], etb[2],
               etw[3], etb[3], etw[4], etb[4], etw[5], etb[5],
               we, ber, sfw, sfb, scw, scb, eqs]
    e_arr, e_ga, e_shm, e_scm, e_gm, sft, scl, geo = pl.pallas_call(
        _edge_pre, grid=(nbe,),
        in_specs=[_bs(prg, RE), _bs(pcg, RE), _bs(edge_attr, RE),
                  _bs(edge_time_emb, RE)] + [_bf(x) for x in ep_full],
        out_specs=[_os(128, RE), _os(16, RE), _os(16, RE), _os(16, RE),
                   _os(16, RE), _os(128, RE), _os(128, RE), _os(128, RE)],
        out_shape=[_sd(E, 128), _sd(E, 16), _sd(E, 16), _sd(E, 16),
                   _sd(E, 16), _sd(E, 128), _sd(E, 128), _sd(E, 128)],
    )(prg, pcg, edge_attr, edge_time_emb, *ep_full)

    qe, kve = _sc_gather([(q, row), (kv, col)])

    wv_arr, w_arr = pl.pallas_call(
        _attn, grid=(nbe,),
        in_specs=[_bs(qe, RE), _bs(kve, RE), _bs(e_arr, RE)],
        out_specs=[_os(128, RE), _os(128, RE)],
        out_shape=[_sd(E, 128), _sd(E, 128)],
    )(qe, kve, e_arr)

    zeros_n = jnp.zeros((N, 128), f32)
    acc1v = _sc_scatter_add(wv_arr, row, zeros_n)
    acc1w = _sc_scatter_add(w_arr, row, zeros_n)

    npo_full = [wo, bor, ff1_w, f1br, ff2_w, f2br]
    hn, hout = pl.pallas_call(
        _node_post, grid=(nbn,),
        in_specs=[_bs(acc1v, RN), _bs(acc1v, RN, off=nbn),
                  _bs(acc1w, RN), _bs(acc1w, RN, off=nbn), _bs(h, RN),
                  _bs(node_mask, RN), _bs(n_ga, RN), _bs(n_shm, RN),
                  _bs(n_scm, RN), _bs(n_gm, RN)] + [_bf(x) for x in npo_full],
        out_specs=[_os(128, RN), _os(128, RN)],
        out_shape=[_sd(N, 128), _sd(N, 128)],
    )(acc1v, acc1v, acc1w, acc1w, h, node_mask, n_ga, n_shm, n_scm, n_gm,
      *npo_full)

    hnr, hnc = _sc_gather([(hn, row), (hn, col)])

    epo_full = [n2e_w, n2br, ff3_w, f3br, ff4_w, f4br]
    (heo,) = pl.pallas_call(
        _edge_post, grid=(nbe,),
        in_specs=[_bs(hnr, RE), _bs(hnc, RE), _bs(edge_attr, RE),
                  _bs(e_ga, RE), _bs(e_shm, RE), _bs(e_scm, RE),
                  _bs(e_gm, RE)] + [_bf(x) for x in epo_full],
        out_specs=[_os(16, RE)],
        out_shape=[_sd(E, 16)],
    )(hnr, hnc, edge_attr, e_ga, e_shm, e_scm, e_gm, *epo_full)

    hor, hoc = _sc_gather([(hout, row), (hout, col)])

    eq_full = [wr_w, wc_w, we2_w, wd_w, eibr, eq_c1_w, c1br, c2r]
    (scat2,) = pl.pallas_call(
        _eqk, grid=(nbe,),
        in_specs=[_bs(hor, RE), _bs(hoc, RE), _bs(heo, RE), _bs(geo, RE),
                  _bs(sft, RE), _bs(scl, RE)] + [_bf(x) for x in eq_full],
        out_specs=[_os(128, RE)],
        out_shape=[_sd(E, 128)],
    )(hor, hoc, heo, geo, sft, scl, *eq_full)

    acc2 = _sc_scatter_add(scat2, row, zeros_n)

    (pos_out,) = pl.pallas_call(
        _posk, grid=(nbn,),
        in_specs=[_bs(pos, RN), _bs(acc2, RN), _bs(acc2, RN, off=nbn)],
        out_specs=[_os(3, RN)],
        out_shape=[_sd(N, 3)],
    )(pos, acc2, acc2)

    return hout, heo, pos_out
